# baseline (speedup 1.0000x reference)
import functools

import jax
import jax.numpy as jnp
from jax import lax
from jax.experimental import pallas as pl
from jax.experimental.pallas import tpu as pltpu
from jax.experimental.pallas import tpu_sc as plsc

N_NODES = 10000
D = 256
HALF = 128
NC = 2
NS = 16
CH = 128
B = 8
NPAD = 10240
ROWS_PER_TILE = NPAD // NS


def _sc_segment_sum(h_split, src3, dst3, n_chunks):
    mesh = plsc.VectorSubcoreMesh(core_axis_name="c", subcore_axis_name="s")
    nb = n_chunks // B
    assert nb % 2 == 0 and nb * B == n_chunks

    @functools.partial(
        pl.kernel,
        mesh=mesh,
        out_type=jax.ShapeDtypeStruct((NC, NPAD, HALF), jnp.float32),
        scratch_types=[
            pltpu.VMEM((B, CH), jnp.int32),
            pltpu.VMEM((B, CH), jnp.int32),
            pltpu.VMEM((B, CH), jnp.int32),
            pltpu.VMEM((B, CH), jnp.int32),
            pltpu.VMEM((CH, HALF), jnp.float32),
            pltpu.VMEM((CH, HALF), jnp.float32),
            pltpu.VMEM_SHARED((NPAD, HALF), jnp.float32),
            pltpu.SemaphoreType.DMA,
            pltpu.SemaphoreType.DMA,
            pltpu.SemaphoreType.DMA,
        ],
    )
    def agg_kernel(h_hbm, src_hbm, dst_hbm, out_hbm, sa, da, sb, db,
                   r0, r1, acc, semg, sems, semi):
        c = lax.axis_index("c")
        s = lax.axis_index("s")
        row0 = s * ROWS_PER_TILE
        bufs = (r0, r1)
        my_src = src_hbm.at[s]
        my_dst = dst_hbm.at[s]
        table = h_hbm.at[c]

        def idx_start(blk, sdst, ddst):
            pltpu.async_copy(my_src.at[pl.ds(blk * B, B)], sdst, semi)
            pltpu.async_copy(my_dst.at[pl.ds(blk * B, B)], ddst, semi)

        def idx_wait():
            pltpu.make_async_copy(my_src.at[pl.ds(0, B)], sa, semi).wait()
            pltpu.make_async_copy(my_dst.at[pl.ds(0, B)], da, semi).wait()

        def gather_start(blk_ref, i, buf):
            pltpu.async_copy(table.at[blk_ref.at[i]], buf, semg)

        def gather_wait():
            pltpu.make_async_copy(table.at[pl.ds(0, CH)], r0, semg).wait()

        def scatter_start(blk_ref, i, buf):
            pltpu.async_copy(buf, acc.at[blk_ref.at[i]], sems, add=True)

        def scatter_wait():
            pltpu.make_async_copy(r0, acc.at[pl.ds(0, CH)], sems).wait()

        idx_start(0, sa, da)

        zero16 = jnp.zeros((16,), jnp.float32)

        def zrow(r, carry):
            for k in range(HALF // 16):
                r0[r, pl.ds(k * 16, 16)] = zero16
            return carry

        lax.fori_loop(0, CH, zrow, 0)

        def zcp(j, carry):
            pltpu.sync_copy(r0, acc.at[pl.ds(row0 + j * CH, CH)])
            return carry

        lax.fori_loop(0, ROWS_PER_TILE // CH, zcp, 0)

        idx_wait()
        idx_start(1, sb, db)
        plsc.subcore_barrier()

        def process(sblk, dblk):
            gather_start(sblk, 0, r0)
            for i in range(B):
                if i >= 1:
                    scatter_wait()
                gather_wait()
                if i + 1 < B:
                    gather_start(sblk, i + 1, bufs[(i + 1) % 2])
                scatter_start(dblk, i, bufs[i % 2])

        def outer(t, carry):
            @pl.when(t >= 1)
            def _():
                idx_wait()
                idx_start(2 * t + 1, sb, db)
                scatter_wait()
            process(sa, da)

            idx_wait()
            @pl.when(t <= nb // 2 - 2)
            def _():
                idx_start(2 * t + 2, sa, da)
            scatter_wait()
            process(sb, db)
            return carry

        lax.fori_loop(0, nb // 2, outer, 0)
        scatter_wait()
        plsc.subcore_barrier()

        pltpu.sync_copy(acc.at[pl.ds(row0, ROWS_PER_TILE)],
                        out_hbm.at[c].at[pl.ds(row0, ROWS_PER_TILE)])

    return agg_kernel(h_split, src3, dst3)


def _tc_linear(agg_split, W, b, relu, split_out):
    BM = 1024
    wr = W.reshape(NC, HALF, D)
    br = b.reshape(1, D)

    def body(a_ref, w_ref, b_ref, o_ref):
        acc = jnp.dot(a_ref[0], w_ref[0], preferred_element_type=jnp.float32)
        acc = acc + jnp.dot(a_ref[1], w_ref[1], preferred_element_type=jnp.float32)
        acc = acc + b_ref[...]
        if relu:
            acc = jnp.maximum(acc, 0.0)
        if split_out:
            o_ref[0] = acc[:, :HALF]
            o_ref[1] = acc[:, HALF:]
        else:
            o_ref[...] = acc

    if split_out:
        out_shape = jax.ShapeDtypeStruct((NC, NPAD, HALF), jnp.float32)
        o_spec = pl.BlockSpec((NC, BM, HALF), lambda i: (0, i, 0))
    else:
        out_shape = jax.ShapeDtypeStruct((NPAD, D), jnp.float32)
        o_spec = pl.BlockSpec((BM, D), lambda i: (i, 0))

    return pl.pallas_call(
        body,
        grid=(NPAD // BM,),
        in_specs=[
            pl.BlockSpec((NC, BM, HALF), lambda i: (0, i, 0)),
            pl.BlockSpec((NC, HALF, D), lambda i: (0, 0, 0)),
            pl.BlockSpec((1, D), lambda i: (0, 0)),
        ],
        out_specs=o_spec,
        out_shape=out_shape,
    )(agg_split, wr, br)


def kernel(feature, edge_index, W1, b1, W2, b2):
    src = edge_index[0].astype(jnp.int32)
    dst = edge_index[1].astype(jnp.int32)
    E = src.shape[0]
    ept = -(-E // NS)
    step = 2 * B * CH
    n_chunks = (-(-ept // step)) * 2 * B
    epad = NS * n_chunks * CH
    src3 = jnp.concatenate(
        [src, jnp.zeros((epad - E,), jnp.int32)]).reshape(NS, n_chunks, CH)
    dst3 = jnp.concatenate(
        [dst, jnp.full((epad - E,), N_NODES, jnp.int32)]).reshape(NS, n_chunks, CH)

    feat_pad = jnp.pad(feature, ((0, NPAD - N_NODES), (0, 0)))
    h_split = feat_pad.reshape(NPAD, NC, HALF).transpose(1, 0, 2)

    agg1 = _sc_segment_sum(h_split, src3, dst3, n_chunks)
    h1 = _tc_linear(agg1, W1, b1, relu=True, split_out=True)
    agg2 = _sc_segment_sum(h1, src3, dst3, n_chunks)
    out = _tc_linear(agg2, W2, b2, relu=False, split_out=False)
    return out[:N_NODES]


# speedup vs baseline: 1.2337x; 1.2337x over previous
import functools

import jax
import jax.numpy as jnp
from jax import lax
from jax.experimental import pallas as pl
from jax.experimental.pallas import tpu as pltpu
from jax.experimental.pallas import tpu_sc as plsc

N_NODES = 10000
D = 256
HALF = 128
NC = 2
NS = 16
CH = 128
NPAD = 10240
ROWS_PER_TILE = NPAD // NS


def _sc_segment_sum(h_split, src3, dst3, n_chunks):
    mesh = plsc.VectorSubcoreMesh(core_axis_name="c", subcore_axis_name="s")

    @functools.partial(
        pl.kernel,
        mesh=mesh,
        out_type=jax.ShapeDtypeStruct((NC, NPAD, HALF), jnp.float32),
        scratch_types=[
            pltpu.VMEM((n_chunks, CH), jnp.int32),
            pltpu.VMEM((n_chunks, CH), jnp.int32),
            pltpu.VMEM((CH, HALF), jnp.float32),
            pltpu.VMEM_SHARED((NPAD, HALF), jnp.float32),
            pltpu.SemaphoreType.DMA,
            pltpu.SemaphoreType.DMA,
        ],
    )
    def agg_kernel(h_hbm, src_hbm, dst_hbm, out_hbm, src_v, dst_v,
                   r0, acc, semg, sems):
        c = lax.axis_index("c")
        s = lax.axis_index("s")
        row0 = s * ROWS_PER_TILE

        zero16 = jnp.zeros((16,), jnp.float32)

        def zrow(r, carry):
            for k in range(HALF // 16):
                r0[r, pl.ds(k * 16, 16)] = zero16
            return carry

        lax.fori_loop(0, CH, zrow, 0)

        def zcp(j, carry):
            pltpu.sync_copy(r0, acc.at[pl.ds(row0 + j * CH, CH)])
            return carry

        lax.fori_loop(0, ROWS_PER_TILE // CH, zcp, 0)
        plsc.subcore_barrier()

        pltpu.sync_copy(src_hbm.at[s], src_v)
        pltpu.sync_copy(dst_hbm.at[s], dst_v)

        table = h_hbm.at[c]

        def step(j, carry):
            pltpu.async_copy(table.at[src_v.at[j]], r0, semg)
            pltpu.make_async_copy(table.at[pl.ds(0, CH)], r0, semg).wait()
            pltpu.async_copy(r0, acc.at[dst_v.at[j]], sems, add=True)
            pltpu.make_async_copy(r0, acc.at[pl.ds(0, CH)], sems).wait()
            return carry

        lax.fori_loop(0, n_chunks, step, 0)
        plsc.subcore_barrier()

        pltpu.sync_copy(acc.at[pl.ds(row0, ROWS_PER_TILE)],
                        out_hbm.at[c].at[pl.ds(row0, ROWS_PER_TILE)])

    return agg_kernel(h_split, src3, dst3)


def _tc_linear(agg_split, W, b, relu, split_out):
    BM = 1024
    wr = W.reshape(NC, HALF, D)
    br = b.reshape(1, D)

    def body(a_ref, w_ref, b_ref, o_ref):
        acc = jnp.dot(a_ref[0], w_ref[0], preferred_element_type=jnp.float32)
        acc = acc + jnp.dot(a_ref[1], w_ref[1], preferred_element_type=jnp.float32)
        acc = acc + b_ref[...]
        if relu:
            acc = jnp.maximum(acc, 0.0)
        if split_out:
            o_ref[0] = acc[:, :HALF]
            o_ref[1] = acc[:, HALF:]
        else:
            o_ref[...] = acc

    if split_out:
        out_shape = jax.ShapeDtypeStruct((NC, NPAD, HALF), jnp.float32)
        o_spec = pl.BlockSpec((NC, BM, HALF), lambda i: (0, i, 0))
    else:
        out_shape = jax.ShapeDtypeStruct((NPAD, D), jnp.float32)
        o_spec = pl.BlockSpec((BM, D), lambda i: (i, 0))

    return pl.pallas_call(
        body,
        grid=(NPAD // BM,),
        in_specs=[
            pl.BlockSpec((NC, BM, HALF), lambda i: (0, i, 0)),
            pl.BlockSpec((NC, HALF, D), lambda i: (0, 0, 0)),
            pl.BlockSpec((1, D), lambda i: (0, 0)),
        ],
        out_specs=o_spec,
        out_shape=out_shape,
    )(agg_split, wr, br)


def kernel(feature, edge_index, W1, b1, W2, b2):
    src = edge_index[0].astype(jnp.int32)
    dst = edge_index[1].astype(jnp.int32)
    E = src.shape[0]
    ept = -(-E // NS)
    n_chunks = -(-ept // CH)
    epad = NS * n_chunks * CH
    src3 = jnp.concatenate(
        [src, jnp.zeros((epad - E,), jnp.int32)]).reshape(NS, n_chunks, CH)
    dst3 = jnp.concatenate(
        [dst, jnp.full((epad - E,), N_NODES, jnp.int32)]).reshape(NS, n_chunks, CH)

    feat_pad = jnp.pad(feature, ((0, NPAD - N_NODES), (0, 0)))
    h_split = feat_pad.reshape(NPAD, NC, HALF).transpose(1, 0, 2)

    agg1 = _sc_segment_sum(h_split, src3, dst3, n_chunks)
    h1 = _tc_linear(agg1, W1, b1, relu=True, split_out=True)
    agg2 = _sc_segment_sum(h1, src3, dst3, n_chunks)
    out = _tc_linear(agg2, W2, b2, relu=False, split_out=False)
    return out[:N_NODES]
